# baseline (speedup 1.0000x reference)
import functools

import jax
import jax.numpy as jnp
from jax import lax
from jax.experimental import pallas as pl
from jax.experimental.pallas import tpu as pltpu
from jax.experimental.pallas import tpu_sc as plsc

VOCAB = 100000
D = 128
NT = 8
LANES = 16


@functools.partial(jax.jit, static_argnums=(2, 3, 4))
def _sc_sum_embed(x_flat, tables_flat, N, NC, NS):
    NW = NC * NS
    TPW = N // NW
    C = 16
    n_chunks = TPW // C
    NQ = n_chunks // 4
    IC = NT * C

    mesh = plsc.VectorSubcoreMesh(core_axis_name="c", subcore_axis_name="s")

    @functools.partial(
        pl.kernel,
        mesh=mesh,
        out_type=jax.ShapeDtypeStruct((N, D), jnp.float32),
        scratch_types=[
            pltpu.VMEM((8, IC), jnp.int32),
            pltpu.VMEM((4, IC, D), jnp.float32),
            pltpu.VMEM((4, C, D), jnp.float32),
            [pltpu.SemaphoreType.DMA] * 8,
            [pltpu.SemaphoreType.DMA] * 4,
            [pltpu.SemaphoreType.DMA] * 4,
        ],
    )
    def k(x_hbm, tables_hbm, out_hbm, idx_v, rows_v, acc_v, sem_i, sem_g,
          sem_o):
        wid = lax.axis_index("s") * NC + lax.axis_index("c")
        wbase = wid * TPW
        offs = (lax.iota(jnp.int32, LANES) & (NT - 1)) * VOCAB

        def fire_idx(c, bi):
            ioff = pl.multiple_of((wbase + c * C) * NT, IC)
            pltpu.async_copy(x_hbm.at[pl.ds(ioff, IC)], idx_v.at[bi],
                             sem_i[bi])

        def fire_gather(c, b, bi):
            ioff = pl.multiple_of((wbase + c * C) * NT, IC)
            pltpu.make_async_copy(x_hbm.at[pl.ds(ioff, IC)], idx_v.at[bi],
                                  sem_i[bi]).wait()
            for j in range(IC // LANES):
                idx_v[bi, pl.ds(j * LANES, LANES)] = (
                    idx_v[bi, pl.ds(j * LANES, LANES)] + offs
                )
            pltpu.async_copy(tables_hbm.at[idx_v.at[bi]], rows_v.at[b],
                             sem_g[b])

        def consume(c, b, bi, drain_pred):
            pltpu.make_async_copy(tables_hbm.at[idx_v.at[bi]], rows_v.at[b],
                                  sem_g[b]).wait()
            if drain_pred is not None:
                @pl.when(drain_pred)
                def _():
                    pltpu.make_async_copy(
                        acc_v.at[b],
                        out_hbm.at[pl.ds(wbase + (c - 4) * C, C)],
                        sem_o[b],
                    ).wait()
            else:
                pltpu.make_async_copy(
                    acc_v.at[b],
                    out_hbm.at[pl.ds(wbase + (c - 4) * C, C)],
                    sem_o[b],
                ).wait()

            @plsc.parallel_loop(0, C, unroll=8)
            def tok_body(t):
                base = t * NT
                for d in range(D // LANES):
                    sl = pl.ds(d * LANES, LANES)
                    s = rows_v[b, base, sl]
                    for r in range(1, NT):
                        s = s + rows_v[b, base + r, sl]
                    acc_v[b, t, sl] = s

            pltpu.async_copy(acc_v.at[b], out_hbm.at[pl.ds(wbase + c * C, C)],
                             sem_o[b])

        for c in range(8):
            fire_idx(c, c)
        for c in range(3):
            fire_gather(c, c, c)

        NO = n_chunks // 8

        def octo_body(o, carry):
            c0 = 8 * o
            for j in range(8):
                c = c0 + j
                b = j & 3
                bi = j
                bg = (j + 3) & 3
                big = (j + 3) & 7
                if j < 5:
                    fire_gather(c + 3, bg, big)
                else:
                    @pl.when(o < NO - 1)
                    def _(c=c, bg=bg, big=big):
                        fire_gather(c + 3, bg, big)
                consume(c, b, bi, drain_pred=(o > 0) if j < 4 else None)
                @pl.when(o < NO - 1)
                def _(c=c, bi=bi):
                    fire_idx(c + 8, bi)
            return carry

        lax.fori_loop(0, NO, octo_body, 0)
        for j in range(4):
            c = n_chunks - 4 + j
            pltpu.make_async_copy(
                acc_v.at[j],
                out_hbm.at[pl.ds(wbase + c * C, C)],
                sem_o[j],
            ).wait()

    return k(x_flat, tables_flat)


def kernel(x, tables):
    B, L, _ = x.shape
    N = B * L
    info = plsc.get_sparse_core_info()
    x_flat = x.reshape(N * NT)
    tables_flat = tables.reshape(NT * VOCAB, D)
    out = _sc_sum_embed(x_flat, tables_flat, N, info.num_cores,
                        info.num_subcores)
    return out.reshape(B, L, D)


# speedup vs baseline: 1.5681x; 1.5681x over previous
import functools

import jax
import jax.numpy as jnp
from jax import lax
from jax.experimental import pallas as pl
from jax.experimental.pallas import tpu as pltpu
from jax.experimental.pallas import tpu_sc as plsc

VOCAB = 100000
D = 128
NT = 8
LANES = 16


@functools.partial(jax.jit, static_argnums=(2, 3, 4))
def _sc_sum_embed(x_flat, tables_flat, N, NC, NS):
    NW = NC * NS
    TPW = N // NW
    C = 16
    n_chunks = TPW // C
    IC = NT * C
    NR = 5
    NI = 10
    NA = 5
    PH = 10
    NO = n_chunks // PH

    mesh = plsc.VectorSubcoreMesh(core_axis_name="c", subcore_axis_name="s")

    @functools.partial(
        pl.kernel,
        mesh=mesh,
        out_type=jax.ShapeDtypeStruct((N, D), jnp.float32),
        scratch_types=[
            pltpu.VMEM((NI, IC), jnp.int32),
            pltpu.VMEM((NR, IC, D), jnp.float32),
            pltpu.VMEM((NA, C, D), jnp.float32),
            [pltpu.SemaphoreType.DMA] * NI,
            [pltpu.SemaphoreType.DMA] * NR,
            [pltpu.SemaphoreType.DMA] * NA,
        ],
    )
    def k(x_hbm, tables_hbm, out_hbm, idx_v, rows_v, acc_v, sem_i, sem_g,
          sem_o):
        wid = lax.axis_index("s") * NC + lax.axis_index("c")
        wbase = wid * TPW
        offs = (lax.iota(jnp.int32, LANES) & (NT - 1)) * VOCAB

        def fire_idx(c, bi):
            ioff = pl.multiple_of((wbase + c * C) * NT, IC)
            pltpu.async_copy(x_hbm.at[pl.ds(ioff, IC)], idx_v.at[bi],
                             sem_i[bi])

        def fire_gather(c, b, bi):
            ioff = pl.multiple_of((wbase + c * C) * NT, IC)
            pltpu.make_async_copy(x_hbm.at[pl.ds(ioff, IC)], idx_v.at[bi],
                                  sem_i[bi]).wait()
            for j in range(IC // LANES):
                idx_v[bi, pl.ds(j * LANES, LANES)] = (
                    idx_v[bi, pl.ds(j * LANES, LANES)] + offs
                )
            pltpu.async_copy(tables_hbm.at[idx_v.at[bi]], rows_v.at[b],
                             sem_g[b])

        def consume(c, b, bi, a, drain_pred):
            pltpu.make_async_copy(tables_hbm.at[idx_v.at[bi]], rows_v.at[b],
                                  sem_g[b]).wait()

            def drain():
                pltpu.make_async_copy(
                    acc_v.at[a],
                    out_hbm.at[pl.ds(wbase + (c - NA) * C, C)],
                    sem_o[a],
                ).wait()

            if drain_pred is None:
                drain()
            else:
                pl.when(drain_pred)(drain)

            @plsc.parallel_loop(0, C, unroll=4)
            def tok_body(t):
                base = t * NT
                for d in range(D // LANES):
                    sl = pl.ds(d * LANES, LANES)
                    s = rows_v[b, base, sl]
                    for r in range(1, NT):
                        s = s + rows_v[b, base + r, sl]
                    acc_v[a, t, sl] = s

            pltpu.async_copy(acc_v.at[a], out_hbm.at[pl.ds(wbase + c * C, C)],
                             sem_o[a])

        for c in range(NI):
            fire_idx(c, c)
        for c in range(4):
            fire_gather(c, c, c)

        def group_body(o, carry):
            c0 = PH * o
            for j in range(PH):
                c = c0 + j
                b = j % NR
                bi = j % NI
                a = j % NA
                bg = (j + 4) % NR
                big = (j + 4) % NI
                if j < PH - 4:
                    fire_gather(c + 4, bg, big)
                else:
                    @pl.when(o < NO - 1)
                    def _(c=c, bg=bg, big=big):
                        fire_gather(c + 4, bg, big)
                consume(c, b, bi, a, drain_pred=(o > 0) if j < NA else None)
                @pl.when(o < NO - 1)
                def _(c=c, bi=bi):
                    fire_idx(c + NI, bi)
            return carry

        lax.fori_loop(0, NO, group_body, 0)
        for j in range(NA):
            c = n_chunks - NA + j
            pltpu.make_async_copy(
                acc_v.at[c % NA],
                out_hbm.at[pl.ds(wbase + c * C, C)],
                sem_o[c % NA],
            ).wait()

    return k(x_flat, tables_flat)


def kernel(x, tables):
    B, L, _ = x.shape
    N = B * L
    info = plsc.get_sparse_core_info()
    x_flat = x.reshape(N * NT)
    tables_flat = tables.reshape(NT * VOCAB, D)
    out = _sc_sum_embed(x_flat, tables_flat, N, info.num_cores,
                        info.num_subcores)
    return out.reshape(B, L, D)


# speedup vs baseline: 1.6034x; 1.0226x over previous
import functools

import jax
import jax.numpy as jnp
from jax import lax
from jax.experimental import pallas as pl
from jax.experimental.pallas import tpu as pltpu
from jax.experimental.pallas import tpu_sc as plsc

VOCAB = 100000
D = 128
NT = 8
LANES = 16


@functools.partial(jax.jit, static_argnums=(2, 3, 4))
def _sc_sum_embed(x_flat, tables_flat, N, NC, NS):
    NW = NC * NS
    TPW = N // NW
    C = 16
    n_chunks = TPW // C
    NQ = n_chunks // 4
    IC = NT * C

    mesh = plsc.VectorSubcoreMesh(core_axis_name="c", subcore_axis_name="s")

    @functools.partial(
        pl.kernel,
        mesh=mesh,
        out_type=jax.ShapeDtypeStruct((N, D), jnp.float32),
        scratch_types=[
            pltpu.VMEM((8, IC), jnp.int32),
            pltpu.VMEM((4, IC, D), jnp.float32),
            pltpu.VMEM((4, C, D), jnp.float32),
            [pltpu.SemaphoreType.DMA] * 8,
            [pltpu.SemaphoreType.DMA] * 4,
            [pltpu.SemaphoreType.DMA] * 4,
        ],
    )
    def k(x_hbm, tables_hbm, out_hbm, idx_v, rows_v, acc_v, sem_i, sem_g,
          sem_o):
        wid = lax.axis_index("s") * NC + lax.axis_index("c")
        wbase = wid * TPW
        offs = (lax.iota(jnp.int32, LANES) & (NT - 1)) * VOCAB

        def fire_idx(c, bi):
            ioff = pl.multiple_of((wbase + c * C) * NT, IC)
            pltpu.async_copy(x_hbm.at[pl.ds(ioff, IC)], idx_v.at[bi],
                             sem_i[bi])

        def fire_gather(c, b, bi):
            ioff = pl.multiple_of((wbase + c * C) * NT, IC)
            pltpu.make_async_copy(x_hbm.at[pl.ds(ioff, IC)], idx_v.at[bi],
                                  sem_i[bi]).wait()
            for j in range(IC // LANES):
                idx_v[bi, pl.ds(j * LANES, LANES)] = (
                    idx_v[bi, pl.ds(j * LANES, LANES)] + offs
                )
            pltpu.async_copy(tables_hbm.at[idx_v.at[bi]], rows_v.at[b],
                             sem_g[b])

        def consume(c, b, bi, drain_pred):
            pltpu.make_async_copy(tables_hbm.at[idx_v.at[bi]], rows_v.at[b],
                                  sem_g[b]).wait()
            if drain_pred is not None:
                @pl.when(drain_pred)
                def _():
                    pltpu.make_async_copy(
                        acc_v.at[b],
                        out_hbm.at[pl.ds(wbase + (c - 4) * C, C)],
                        sem_o[b],
                    ).wait()
            else:
                pltpu.make_async_copy(
                    acc_v.at[b],
                    out_hbm.at[pl.ds(wbase + (c - 4) * C, C)],
                    sem_o[b],
                ).wait()

            @plsc.parallel_loop(0, C, unroll=4)
            def tok_body(t):
                base = t * NT
                for d in range(D // LANES):
                    sl = pl.ds(d * LANES, LANES)
                    s = rows_v[b, base, sl]
                    for r in range(1, NT):
                        s = s + rows_v[b, base + r, sl]
                    acc_v[b, t, sl] = s

            pltpu.async_copy(acc_v.at[b], out_hbm.at[pl.ds(wbase + c * C, C)],
                             sem_o[b])

        for c in range(8):
            fire_idx(c, c)
        for c in range(3):
            fire_gather(c, c, c)

        NO = n_chunks // 8

        def octo_body(o, carry):
            c0 = 8 * o
            for j in range(8):
                c = c0 + j
                b = j & 3
                bi = j
                bg = (j + 3) & 3
                big = (j + 3) & 7
                if j < 5:
                    fire_gather(c + 3, bg, big)
                else:
                    @pl.when(o < NO - 1)
                    def _(c=c, bg=bg, big=big):
                        fire_gather(c + 3, bg, big)
                consume(c, b, bi, drain_pred=(o > 0) if j < 4 else None)
                @pl.when(o < NO - 1)
                def _(c=c, bi=bi):
                    fire_idx(c + 8, bi)
            return carry

        lax.fori_loop(0, NO, octo_body, 0)
        for j in range(4):
            c = n_chunks - 4 + j
            pltpu.make_async_copy(
                acc_v.at[j],
                out_hbm.at[pl.ds(wbase + c * C, C)],
                sem_o[j],
            ).wait()

    return k(x_flat, tables_flat)


def kernel(x, tables):
    B, L, _ = x.shape
    N = B * L
    info = plsc.get_sparse_core_info()
    x_flat = x.reshape(N * NT)
    tables_flat = tables.reshape(NT * VOCAB, D)
    out = _sc_sum_embed(x_flat, tables_flat, N, info.num_cores,
                        info.num_subcores)
    return out.reshape(B, L, D)
